# Initial kernel scaffold; baseline (speedup 1.0000x reference)
import functools

import jax
import jax.numpy as jnp
from jax import lax
from jax.experimental import pallas as pl
from jax.experimental.pallas import tpu as pltpu
from jax.experimental.pallas import tpu_sc as plsc

_N = 10000
_E = 320000
_D = 128
_L = 5
_G = 256
_EPS = 1e-5

_NC = 2
_NS = 16
_NW = _NC * _NS
_K = 80
_EC = _E // (_NW * _K)
_RPT = _N // _NS
_NPAD = 10240
_XC = _NPAD // (_NW * _K)

_R = 1000
_NB = _N // _R

_mesh = plsc.VectorSubcoreMesh(core_axis_name="c", subcore_axis_name="s")


def _make_edge_scatter(w):
    @functools.partial(
        pl.kernel,
        mesh=_mesh,
        out_type=jax.ShapeDtypeStruct((_NC, _N, w), jnp.float32),
        scratch_types=[
            pltpu.VMEM((_EC, _K), jnp.int32),
            pltpu.VMEM((_EC, _K), jnp.int32),
            pltpu.VMEM((_K, w), jnp.float32),
            pltpu.VMEM((_RPT, w), jnp.float32),
            pltpu.VMEM_SHARED((_N, w), jnp.float32),
            pltpu.SemaphoreType.DMA,
        ],
    )
    def k(table_hbm, src_hbm, dst_hbm, zeros_hbm, out_hbm,
          src_v, dst_v, rows_v, buf_v, acc_sh, sem):
        c = lax.axis_index("c")
        s = lax.axis_index("s")
        wid = c * _NS + s
        row0 = s * _RPT
        pltpu.sync_copy(zeros_hbm, buf_v)
        pltpu.sync_copy(buf_v, acc_sh.at[pl.ds(row0, _RPT)])
        pltpu.sync_copy(src_hbm.at[wid], src_v)
        pltpu.sync_copy(dst_hbm.at[wid], dst_v)
        plsc.subcore_barrier()

        def body(j, carry):
            pltpu.async_copy(table_hbm.at[src_v.at[j]], rows_v, sem).wait()
            pltpu.sync_copy(rows_v, acc_sh.at[dst_v.at[j]], add=True)
            return carry

        lax.fori_loop(0, _EC, body, 0)
        plsc.subcore_barrier()
        pltpu.sync_copy(acc_sh.at[pl.ds(row0, _RPT)], buf_v)
        pltpu.sync_copy(buf_v, out_hbm.at[c, pl.ds(row0, _RPT)])

    return k


_edge_scatter_d = _make_edge_scatter(_D)
_edge_scatter_c = _make_edge_scatter(32)


@functools.partial(
    pl.kernel,
    mesh=_mesh,
    out_type=jax.ShapeDtypeStruct((_NPAD, _D), jnp.float32),
    scratch_types=[
        pltpu.VMEM((_XC, _K), jnp.int32),
        pltpu.VMEM((_K, _D), jnp.float32),
        pltpu.SemaphoreType.DMA,
    ],
)
def _embed_gather(table_hbm, idx_hbm, out_hbm, idx_v, rows_v, sem):
    c = lax.axis_index("c")
    s = lax.axis_index("s")
    wid = c * _NS + s
    pltpu.sync_copy(idx_hbm.at[wid], idx_v)

    def body(j, carry):
        pltpu.async_copy(table_hbm.at[idx_v.at[j]], rows_v, sem).wait()
        base = pl.multiple_of(wid * (_XC * _K) + j * _K, 8)
        pltpu.sync_copy(rows_v, out_hbm.at[pl.ds(base, _K)])
        return carry

    lax.fori_loop(0, _XC, body, 0)


def _dense_a(p0_ref, p1_ref, h_ref, cnt0_ref, cnt1_ref, et_ref, cb_ref,
             w1_ref, b1_ref, w2_ref, b2_ref, ho_ref, stats_ref, acc_ref):
    i = pl.program_id(0)
    cnt = cnt0_ref[...] + cnt1_ref[...]
    a = (p0_ref[...] + p1_ref[...] + h_ref[...] + cb_ref[...]
         + jnp.dot(cnt, et_ref[...], preferred_element_type=jnp.float32))
    hm = jnp.maximum(
        jnp.dot(a, w1_ref[...], preferred_element_type=jnp.float32)
        + b1_ref[...], 0.0)
    ho = (jnp.dot(hm, w2_ref[...], preferred_element_type=jnp.float32)
          + b2_ref[...])
    ho_ref[...] = ho

    @pl.when(i == 0)
    def _():
        acc_ref[...] = jnp.zeros_like(acc_ref)

    acc_ref[0:1, :] = acc_ref[0:1, :] + jnp.sum(ho, axis=0, keepdims=True)
    acc_ref[1:2, :] = acc_ref[1:2, :] + jnp.sum(ho * ho, axis=0, keepdims=True)

    @pl.when(i == _NB - 1)
    def _():
        stats_ref[...] = acc_ref[...]


def _dense_a_call(p0, p1, h, cnt0, cnt1, et, cb, w1, b1, w2, b2):
    blk = lambda r, c: pl.BlockSpec((r, c), lambda i: (i, 0))
    cst = lambda r, c: pl.BlockSpec((r, c), lambda i: (0, 0))
    return pl.pallas_call(
        _dense_a,
        grid=(_NB,),
        in_specs=[
            blk(_R, _D), blk(_R, _D), blk(_R, _D),
            blk(_R, 32), blk(_R, 32),
            cst(32, _D), cst(1, _D),
            cst(_D, 2 * _D), cst(1, 2 * _D), cst(2 * _D, _D), cst(1, _D),
        ],
        out_specs=[blk(_R, _D), cst(8, _D)],
        out_shape=[
            jax.ShapeDtypeStruct((_N, _D), jnp.float32),
            jax.ShapeDtypeStruct((8, _D), jnp.float32),
        ],
        scratch_shapes=[pltpu.VMEM((8, _D), jnp.float32)],
    )(p0, p1, h, cnt0, cnt1, et, cb, w1, b1, w2, b2)


def _dense_b(last, ho_ref, stats_ref, g_ref, b_ref, out_ref):
    mean = stats_ref[0:1, :] / _N
    var = stats_ref[1:2, :] / _N - mean * mean
    inv = lax.rsqrt(var + _EPS)
    y = (ho_ref[...] - mean) * inv * g_ref[...] + b_ref[...]
    if not last:
        y = jnp.maximum(y, 0.0)
    out_ref[...] = y


def _dense_b_call(last, ho, stats, g, b):
    return pl.pallas_call(
        functools.partial(_dense_b, last),
        grid=(_NB,),
        in_specs=[
            pl.BlockSpec((_R, _D), lambda i: (i, 0)),
            pl.BlockSpec((8, _D), lambda i: (0, 0)),
            pl.BlockSpec((1, _D), lambda i: (0, 0)),
            pl.BlockSpec((1, _D), lambda i: (0, 0)),
        ],
        out_specs=pl.BlockSpec((_R, _D), lambda i: (i, 0)),
        out_shape=jax.ShapeDtypeStruct((_N, _D), jnp.float32),
    )(ho, stats, g, b)


def _pool(h_ref, batch_ref, pw_ref, pb_ref, out_ref, acc_ref):
    i = pl.program_id(0)

    @pl.when(i == 0)
    def _():
        acc_ref[...] = jnp.zeros_like(acc_ref)

    seg = lax.broadcasted_iota(jnp.int32, (_R, _G), 1)
    onehot = (batch_ref[...] == seg).astype(jnp.float32)
    acc_ref[...] = acc_ref[...] + lax.dot_general(
        onehot, h_ref[...], (((0,), (0,)), ((), ())),
        preferred_element_type=jnp.float32)

    @pl.when(i == _NB - 1)
    def _():
        out_ref[...] = (jnp.dot(acc_ref[...], pw_ref[...],
                                preferred_element_type=jnp.float32)
                        + pb_ref[...])


def _pool_call(h, batch2, pwpad, pb):
    return pl.pallas_call(
        _pool,
        grid=(_NB,),
        in_specs=[
            pl.BlockSpec((_R, _D), lambda i: (i, 0)),
            pl.BlockSpec((_R, 1), lambda i: (i, 0)),
            pl.BlockSpec((_D, _D), lambda i: (0, 0)),
            pl.BlockSpec((1, _D), lambda i: (0, 0)),
        ],
        out_specs=pl.BlockSpec((_G, _D), lambda i: (0, 0)),
        out_shape=jax.ShapeDtypeStruct((_G, _D), jnp.float32),
        scratch_shapes=[pltpu.VMEM((_G, _D), jnp.float32)],
    )(h, batch2, pwpad, pb)


def kernel(x, edge_index, edge_attr, batch, x_emb1, x_emb2, edge_emb1,
           edge_emb2, W1, b1, W2, b2, bn_g, bn_b, pred_W, pred_b):
    f32 = jnp.float32
    src = edge_index[0].astype(jnp.int32).reshape(_NW, _EC, _K)
    dst = edge_index[1].astype(jnp.int32).reshape(_NW, _EC, _K)
    ecombo = (edge_attr[:, 0] * 3 + edge_attr[:, 1]).astype(jnp.int32)
    ecombo = ecombo.reshape(_NW, _EC, _K)
    xc = (x[:, 0] * 3 + x[:, 1]).astype(jnp.int32)
    xc = jnp.concatenate([xc, jnp.zeros((_NPAD - _N,), jnp.int32)])
    xc = xc.reshape(_NW, _XC, _K)
    t_node = (x_emb1[:, None, :] + x_emb2[None, :, :]).reshape(-1, _D)
    t_node = t_node.astype(f32)
    eye32 = jnp.eye(32, dtype=f32)
    tcb = jnp.arange(18) // 3
    rcb = jnp.arange(18) % 3
    et = edge_emb1[:, tcb, :] + edge_emb2[:, rcb, :]
    et = jnp.concatenate(
        [et, jnp.zeros((_L, 32 - 18, _D), f32)], axis=1)
    cb = (edge_emb1[:, 4, :] + edge_emb2[:, 0, :]).reshape(_L, 1, _D)
    zeros_d = jnp.zeros((_RPT, _D), f32)
    zeros_c = jnp.zeros((_RPT, 32), f32)

    cnt = _edge_scatter_c(eye32, ecombo, dst, zeros_c)
    h = _embed_gather(t_node, xc)[:_N]

    for l in range(_L):
        p = _edge_scatter_d(h, src, dst, zeros_d)
        ho, stats = _dense_a_call(
            p[0], p[1], h, cnt[0], cnt[1], et[l], cb[l],
            W1[l], b1[l].reshape(1, -1), W2[l], b2[l].reshape(1, -1))
        h = _dense_b_call(l == _L - 1, ho, stats,
                          bn_g[l].reshape(1, -1), bn_b[l].reshape(1, -1))

    pwpad = jnp.pad(pred_W.astype(f32), ((0, 0), (0, _D - pred_W.shape[1])))
    pbb = jnp.broadcast_to(pred_b.reshape(1, -1), (1, _D)).astype(f32)
    out = _pool_call(h, batch.astype(jnp.int32).reshape(-1, 1), pwpad, pbb)
    return out[:, :pred_W.shape[1]]


# speedup vs baseline: 4.9297x; 4.9297x over previous
import functools

import jax
import jax.numpy as jnp
from jax import lax
from jax.experimental import pallas as pl
from jax.experimental.pallas import tpu as pltpu
from jax.experimental.pallas import tpu_sc as plsc

_N = 10000
_E = 320000
_D = 128
_L = 5
_G = 256
_EPS = 1e-5

_NC = 2
_NS = 16
_NW = _NC * _NS
_K = 80
_EC = _E // (_NW * _K)
_NPAD = 10240
_RPT = _NPAD // _NS
_XC = _NPAD // (_NW * _K)

_R = 1000
_NB = _N // _R

_mesh = plsc.VectorSubcoreMesh(core_axis_name="c", subcore_axis_name="s")


def _make_edge_scatter(w):
    @functools.partial(
        pl.kernel,
        mesh=_mesh,
        out_type=jax.ShapeDtypeStruct((_NC, _NPAD, w), jnp.float32),
        scratch_types=[
            pltpu.VMEM((_EC, _K), jnp.int32),
            pltpu.VMEM((_EC, _K), jnp.int32),
            pltpu.VMEM((_K, w), jnp.float32),
            pltpu.VMEM_SHARED((_NPAD, w), jnp.float32),
            pltpu.SemaphoreType.DMA,
        ],
    )
    def k(table_hbm, src_hbm, dst_hbm, zeros_hbm, out_hbm,
          src_v, dst_v, rows_v, acc_sh, sem):
        c = lax.axis_index("c")
        s = lax.axis_index("s")
        wid = c * _NS + s
        row0 = pl.multiple_of(s * _RPT, 8)
        pltpu.sync_copy(zeros_hbm, acc_sh.at[pl.ds(row0, _RPT)])
        pltpu.sync_copy(src_hbm.at[wid], src_v)
        pltpu.sync_copy(dst_hbm.at[wid], dst_v)
        plsc.subcore_barrier()

        def body(j, carry):
            pltpu.async_copy(table_hbm.at[src_v.at[j]], rows_v, sem).wait()
            pltpu.sync_copy(rows_v, acc_sh.at[dst_v.at[j]], add=True)
            return carry

        lax.fori_loop(0, _EC, body, 0)
        plsc.subcore_barrier()
        pltpu.sync_copy(acc_sh.at[pl.ds(row0, _RPT)],
                        out_hbm.at[c, pl.ds(row0, _RPT)])

    return k


_edge_scatter = _make_edge_scatter(_D)


@functools.partial(
    pl.kernel,
    mesh=_mesh,
    out_type=jax.ShapeDtypeStruct((_NPAD, _D), jnp.float32),
    scratch_types=[
        pltpu.VMEM((_XC, _K), jnp.int32),
        pltpu.VMEM((_K, _D), jnp.float32),
        pltpu.SemaphoreType.DMA,
    ],
)
def _embed_gather(table_hbm, idx_hbm, out_hbm, idx_v, rows_v, sem):
    c = lax.axis_index("c")
    s = lax.axis_index("s")
    wid = c * _NS + s
    pltpu.sync_copy(idx_hbm.at[wid], idx_v)

    def body(j, carry):
        pltpu.async_copy(table_hbm.at[idx_v.at[j]], rows_v, sem).wait()
        base = pl.multiple_of(wid * (_XC * _K) + j * _K, 8)
        pltpu.sync_copy(rows_v, out_hbm.at[pl.ds(base, _K)])
        return carry

    lax.fori_loop(0, _XC, body, 0)


def _dense_a(p_ref, h_ref, cnt_ref, et_ref, cb_ref,
             w1_ref, b1_ref, w2_ref, b2_ref, ho_ref, stats_ref, acc_ref):
    i = pl.program_id(0)
    cnt = cnt_ref[0] + cnt_ref[1]
    a = (p_ref[0] + p_ref[1] + h_ref[...] + cb_ref[...]
         + jnp.dot(cnt, et_ref[...], preferred_element_type=jnp.float32,
                   precision=lax.Precision.HIGHEST))
    hm = jnp.maximum(
        jnp.dot(a, w1_ref[...], preferred_element_type=jnp.float32)
        + b1_ref[...], 0.0)
    ho = (jnp.dot(hm, w2_ref[...], preferred_element_type=jnp.float32)
          + b2_ref[...])
    ho_ref[...] = ho

    @pl.when(i == 0)
    def _():
        acc_ref[...] = jnp.zeros_like(acc_ref)

    acc_ref[0:1, :] = acc_ref[0:1, :] + jnp.sum(ho, axis=0, keepdims=True)

    @pl.when(i == _NB - 1)
    def _():
        stats_ref[...] = acc_ref[...]


def _dense_a_call(p, h, cnt, et, cb, w1, b1, w2, b2):
    blk = lambda r, c: pl.BlockSpec((r, c), lambda i: (i, 0))
    cst = lambda r, c: pl.BlockSpec((r, c), lambda i: (0, 0))
    return pl.pallas_call(
        _dense_a,
        grid=(_NB,),
        in_specs=[
            pl.BlockSpec((2, _R, _D), lambda i: (0, i, 0)),
            blk(_R, _D),
            pl.BlockSpec((2, _R, _D), lambda i: (0, i, 0)),
            cst(_D, _D), cst(1, _D),
            cst(_D, 2 * _D), cst(1, 2 * _D), cst(2 * _D, _D), cst(1, _D),
        ],
        out_specs=[blk(_R, _D), cst(8, _D)],
        out_shape=[
            jax.ShapeDtypeStruct((_N, _D), jnp.float32),
            jax.ShapeDtypeStruct((8, _D), jnp.float32),
        ],
        scratch_shapes=[pltpu.VMEM((8, _D), jnp.float32)],
    )(p, h, cnt, et, cb, w1, b1, w2, b2)


def _dense_v(ho_ref, stats_ref, vstats_ref, acc_ref):
    i = pl.program_id(0)
    mean = stats_ref[0:1, :] / _N
    dev = ho_ref[...] - mean

    @pl.when(i == 0)
    def _():
        acc_ref[...] = jnp.zeros_like(acc_ref)

    acc_ref[0:1, :] = acc_ref[0:1, :] + jnp.sum(dev * dev, axis=0,
                                                keepdims=True)

    @pl.when(i == _NB - 1)
    def _():
        vstats_ref[...] = acc_ref[...]


def _dense_v_call(ho, stats):
    return pl.pallas_call(
        _dense_v,
        grid=(_NB,),
        in_specs=[
            pl.BlockSpec((_R, _D), lambda i: (i, 0)),
            pl.BlockSpec((8, _D), lambda i: (0, 0)),
        ],
        out_specs=pl.BlockSpec((8, _D), lambda i: (0, 0)),
        out_shape=jax.ShapeDtypeStruct((8, _D), jnp.float32),
        scratch_shapes=[pltpu.VMEM((8, _D), jnp.float32)],
    )(ho, stats)


def _dense_b(last, ho_ref, stats_ref, vstats_ref, g_ref, b_ref, out_ref):
    mean = stats_ref[0:1, :] / _N
    var = vstats_ref[0:1, :] / _N
    inv = lax.rsqrt(var + _EPS)
    y = (ho_ref[...] - mean) * inv * g_ref[...] + b_ref[...]
    if not last:
        y = jnp.maximum(y, 0.0)
    out_ref[...] = y


def _dense_b_call(last, ho, stats, vstats, g, b):
    return pl.pallas_call(
        functools.partial(_dense_b, last),
        grid=(_NB,),
        in_specs=[
            pl.BlockSpec((_R, _D), lambda i: (i, 0)),
            pl.BlockSpec((8, _D), lambda i: (0, 0)),
            pl.BlockSpec((8, _D), lambda i: (0, 0)),
            pl.BlockSpec((1, _D), lambda i: (0, 0)),
            pl.BlockSpec((1, _D), lambda i: (0, 0)),
        ],
        out_specs=pl.BlockSpec((_R, _D), lambda i: (i, 0)),
        out_shape=jax.ShapeDtypeStruct((_N, _D), jnp.float32),
    )(ho, stats, vstats, g, b)


def _pool(h_ref, batch_ref, pw_ref, pb_ref, out_ref, acc_ref):
    i = pl.program_id(0)

    @pl.when(i == 0)
    def _():
        acc_ref[...] = jnp.zeros_like(acc_ref)

    seg = lax.broadcasted_iota(jnp.int32, (_R, _G), 1)
    onehot = (batch_ref[...] == seg).astype(jnp.float32)
    acc_ref[...] = acc_ref[...] + lax.dot_general(
        onehot, h_ref[...], (((0,), (0,)), ((), ())),
        preferred_element_type=jnp.float32,
        precision=lax.Precision.HIGHEST)

    @pl.when(i == _NB - 1)
    def _():
        out_ref[...] = (jnp.dot(acc_ref[...], pw_ref[...],
                                preferred_element_type=jnp.float32)
                        + pb_ref[...])


def _pool_call(h, batch2, pwpad, pb):
    return pl.pallas_call(
        _pool,
        grid=(_NB,),
        in_specs=[
            pl.BlockSpec((_R, _D), lambda i: (i, 0)),
            pl.BlockSpec((_R, 1), lambda i: (i, 0)),
            pl.BlockSpec((_D, _D), lambda i: (0, 0)),
            pl.BlockSpec((1, _D), lambda i: (0, 0)),
        ],
        out_specs=pl.BlockSpec((_G, _D), lambda i: (0, 0)),
        out_shape=jax.ShapeDtypeStruct((_G, _D), jnp.float32),
        scratch_shapes=[pltpu.VMEM((_G, _D), jnp.float32)],
    )(h, batch2, pwpad, pb)


def kernel(x, edge_index, edge_attr, batch, x_emb1, x_emb2, edge_emb1,
           edge_emb2, W1, b1, W2, b2, bn_g, bn_b, pred_W, pred_b):
    f32 = jnp.float32
    src = edge_index[0].astype(jnp.int32).reshape(_NW, _EC, _K)
    dst = edge_index[1].astype(jnp.int32).reshape(_NW, _EC, _K)
    ecombo = (edge_attr[:, 0] * 3 + edge_attr[:, 1]).astype(jnp.int32)
    ecombo = ecombo.reshape(_NW, _EC, _K)
    xc = (x[:, 0] * 3 + x[:, 1]).astype(jnp.int32)
    xc = jnp.concatenate([xc, jnp.zeros((_NPAD - _N,), jnp.int32)])
    xc = xc.reshape(_NW, _XC, _K)
    t_node = (x_emb1[:, None, :] + x_emb2[None, :, :]).reshape(-1, _D)
    t_node = t_node.astype(f32)
    eye_tab = jnp.eye(32, _D, dtype=f32)
    tcb = jnp.arange(18) // 3
    rcb = jnp.arange(18) % 3
    et = edge_emb1[:, tcb, :] + edge_emb2[:, rcb, :]
    et = jnp.concatenate(
        [et, jnp.zeros((_L, _D - 18, _D), f32)], axis=1)
    cb = (edge_emb1[:, 4, :] + edge_emb2[:, 0, :]).reshape(_L, 1, _D)
    zeros_d = jnp.zeros((_RPT, _D), f32)

    cnt = _edge_scatter(eye_tab, ecombo, dst, zeros_d)
    h = _embed_gather(t_node, xc)

    for l in range(_L):
        p = _edge_scatter(h, src, dst, zeros_d)
        ho, stats = _dense_a_call(
            p, h, cnt, et[l], cb[l],
            W1[l], b1[l].reshape(1, -1), W2[l], b2[l].reshape(1, -1))
        vstats = _dense_v_call(ho, stats)
        h = _dense_b_call(l == _L - 1, ho, stats, vstats,
                          bn_g[l].reshape(1, -1), bn_b[l].reshape(1, -1))

    pwpad = jnp.pad(pred_W.astype(f32), ((0, 0), (0, _D - pred_W.shape[1])))
    pbb = jnp.broadcast_to(pred_b.reshape(1, -1), (1, _D)).astype(f32)
    out = _pool_call(h, batch.astype(jnp.int32).reshape(-1, 1), pwpad, pbb)
    return out[:, :pred_W.shape[1]]
